# Initial kernel scaffold; baseline (speedup 1.0000x reference)
import jax
import jax.numpy as jnp
from jax.experimental import pallas as pl


def kernel(x, edge_index, batch, embed, W1, b1, W2, b2, lin1_W, lin1_b, lin2_W, lin2_b):
    raise NotImplementedError("write your pallas kernel here")


# speedup vs baseline: 1.8920x; 1.8920x over previous
import functools

import jax
import jax.numpy as jnp
from jax import lax
from jax.experimental import pallas as pl
from jax.experimental.pallas import tpu as pltpu

N = 10000
E = 160000
D_EMB = 200
H = 2000
B = 64

_BM = 400


def _mm_body(x_ref, w_ref, b_ref, pre_ref, post_ref, o_ref, *, leaky, use_pre,
             use_post):
    x = x_ref[...]
    if use_pre:
        x = x * pre_ref[...]
    z = jnp.dot(x, w_ref[...], preferred_element_type=jnp.float32)
    z = z + b_ref[...]
    if leaky:
        z = jnp.where(z > 0, z, 0.01 * z)
    if use_post:
        z = z * post_ref[...]
    o_ref[...] = z


def _mm(x, w, b, pre=None, post=None, leaky=True):
    m, k = x.shape
    _, n = w.shape
    use_pre = pre is not None
    use_post = post is not None
    if pre is None:
        pre = jnp.zeros((m, 1), jnp.float32)
    if post is None:
        post = jnp.zeros((m, 1), jnp.float32)
    grid = (m // _BM,)
    return pl.pallas_call(
        functools.partial(_mm_body, leaky=leaky, use_pre=use_pre,
                          use_post=use_post),
        grid=grid,
        in_specs=[
            pl.BlockSpec((_BM, k), lambda i: (i, 0)),
            pl.BlockSpec((k, n), lambda i: (0, 0)),
            pl.BlockSpec((1, n), lambda i: (0, 0)),
            pl.BlockSpec((_BM, 1), lambda i: (i, 0)),
            pl.BlockSpec((_BM, 1), lambda i: (i, 0)),
        ],
        out_specs=pl.BlockSpec((_BM, n), lambda i: (i, 0)),
        out_shape=jax.ShapeDtypeStruct((m, n), jnp.float32),
    )(x, w, b.reshape(1, n), pre, post)


def _head_body(g_ref, w1_ref, b1_ref, w2_ref, b2_ref, o_ref):
    g = g_ref[...]
    g = jnp.where(jnp.isfinite(g), g, 0.0)
    z = jnp.dot(g, w1_ref[...], preferred_element_type=jnp.float32) + b1_ref[...]
    z = jnp.where(z > 0, z, 0.01 * z)
    z = jnp.dot(z, w2_ref[...], preferred_element_type=jnp.float32) + b2_ref[...]
    o_ref[...] = jnp.where(z > 0, z, 0.01 * z)


def _head(g, lin1_W, lin1_b, lin2_W, lin2_b):
    return pl.pallas_call(
        _head_body,
        out_shape=jax.ShapeDtypeStruct((B, 4), jnp.float32),
    )(g, lin1_W, lin1_b.reshape(1, -1), lin2_W, lin2_b.reshape(1, -1))


def kernel(x, edge_index, batch, embed, W1, b1, W2, b2, lin1_W, lin1_b,
           lin2_W, lin2_b):
    loop = jnp.arange(N, dtype=edge_index.dtype)
    src = jnp.concatenate([edge_index[0], loop])
    dst = jnp.concatenate([edge_index[1], loop])

    deg = jnp.zeros((N,), jnp.float32).at[dst].add(1.0)
    dinv = jax.lax.rsqrt(jnp.maximum(deg, 1.0)).reshape(N, 1)

    h0 = embed[x]
    s0 = h0 * dinv
    agg1 = jnp.zeros((N, D_EMB), jnp.float32).at[dst].add(s0[src])
    s1 = _mm(agg1, W1, b1, pre=dinv, post=dinv, leaky=True)

    agg2 = jnp.zeros((N, H), jnp.float32).at[dst].add(s1[src])
    h2 = _mm(agg2, W2, b2, pre=dinv, post=None, leaky=True)

    g = jax.ops.segment_max(h2, batch, num_segments=B)
    return _head(g, lin1_W, lin1_b, lin2_W, lin2_b)


# speedup vs baseline: 3.0398x; 1.6067x over previous
import functools

import jax
import jax.numpy as jnp
from jax import lax
from jax.experimental import pallas as pl
from jax.experimental.pallas import tpu as pltpu
from jax.experimental.pallas import tpu_sc as plsc

N = 10000
E = 160000
D_EMB = 200
H = 2000
B = 64

NP = 10240
ACC_ROWS = 10368
ZPT = ACC_ROWS // 16
EP = 172032
EPW = EP // 32
TB = 128
NB = EPW // TB
CHUNK = 128
D_PAD = 256
H_PAD = 2048

_BM = 400

_mesh = plsc.VectorSubcoreMesh(core_axis_name="c", subcore_axis_name="s")


def _sc_degree_body(dst_hbm, ones_hbm, zeros_hbm, deg_a, deg_b, acc, onesb,
                    didx):
    cid = lax.axis_index("c")
    sid = lax.axis_index("s")
    wid = sid * 2 + cid
    z0 = sid * ZPT
    r0 = sid * 640
    pltpu.sync_copy(zeros_hbm.at[pl.ds(z0, ZPT)], acc.at[pl.ds(z0, ZPT)])
    pltpu.sync_copy(ones_hbm, onesb)
    plsc.subcore_barrier()
    ebase = wid * EPW

    def body(e, c):
        pltpu.sync_copy(dst_hbm.at[pl.ds(ebase + e * TB, TB)], didx)
        pltpu.sync_copy(onesb, acc.at[didx], add=True)
        return c

    lax.fori_loop(0, NB, body, 0)
    plsc.subcore_barrier()

    @pl.when(cid == 0)
    def _():
        pltpu.sync_copy(acc.at[pl.ds(r0, 640)], deg_a.at[pl.ds(r0, 640)])

    @pl.when(cid == 1)
    def _():
        pltpu.sync_copy(acc.at[pl.ds(r0, 640)], deg_b.at[pl.ds(r0, 640)])


def _sc_degree(dst_pad, ones128, zeros128):
    k = pl.kernel(
        _sc_degree_body,
        out_type=[jax.ShapeDtypeStruct((NP, CHUNK), jnp.float32),
                  jax.ShapeDtypeStruct((NP, CHUNK), jnp.float32)],
        mesh=_mesh,
        scratch_types=[pltpu.VMEM_SHARED((ACC_ROWS, CHUNK), jnp.float32),
                       pltpu.VMEM((TB, CHUNK), jnp.float32),
                       pltpu.VMEM((TB,), jnp.int32)],
    )
    return k(dst_pad, ones128, zeros128)


def _sc_gather_body(table, idx_hbm, out, buf, idxb, sem):
    cid = lax.axis_index("c")
    sid = lax.axis_index("s")
    wid = sid * 2 + cid
    r0 = wid * 320
    for e in range(5):
        base = r0 + e * 64
        pltpu.sync_copy(idx_hbm.at[pl.ds(base, 64)], idxb)
        pltpu.async_copy(table.at[idxb], buf, sem).wait()
        pltpu.sync_copy(buf, out.at[pl.ds(base, 64)])


def _sc_gather(table, idx):
    k = pl.kernel(
        _sc_gather_body,
        out_type=jax.ShapeDtypeStruct((NP, D_PAD), jnp.float32),
        mesh=_mesh,
        scratch_types=[pltpu.VMEM((64, D_PAD), jnp.float32),
                       pltpu.VMEM((64,), jnp.int32),
                       pltpu.SemaphoreType.DMA],
    )
    return k(table, idx)


def _sc_segsum_body(table, srck, dst_hbm, zeros_hbm, out_a, out_b, acc, gbuf,
                    sidx, didx, sem, *, nchunk, width):
    cid = lax.axis_index("c")
    sid = lax.axis_index("s")
    wid = sid * 2 + cid
    ebase = wid * EPW
    z0 = sid * ZPT
    r0 = sid * 640
    for j in range(nchunk):
        pltpu.sync_copy(zeros_hbm.at[pl.ds(z0, ZPT)], acc.at[pl.ds(z0, ZPT)])
        plsc.subcore_barrier()

        def body(e, c):
            b0 = ebase + e * TB
            pltpu.sync_copy(srck.at[j, 0, pl.ds(b0, TB)], sidx)
            pltpu.sync_copy(dst_hbm.at[pl.ds(b0, TB)], didx)
            pltpu.async_copy(table.at[sidx], gbuf, sem).wait()
            pltpu.sync_copy(gbuf, acc.at[didx], add=True)
            return c

        lax.fori_loop(0, NB, body, 0)
        plsc.subcore_barrier()

        @pl.when(cid == 0)
        def _():
            pltpu.sync_copy(acc.at[pl.ds(r0, 640)],
                            out_a.at[j, pl.ds(r0, 640)])

        @pl.when(cid == 1)
        def _():
            pltpu.sync_copy(acc.at[pl.ds(r0, 640)],
                            out_b.at[j, pl.ds(r0, 640)])

        plsc.subcore_barrier()


def _sc_segsum(table, srck, dst_pad, zerosC, nchunk):
    k = pl.kernel(
        functools.partial(_sc_segsum_body, nchunk=nchunk, width=CHUNK),
        out_type=[jax.ShapeDtypeStruct((nchunk, NP, CHUNK), jnp.float32),
                  jax.ShapeDtypeStruct((nchunk, NP, CHUNK), jnp.float32)],
        mesh=_mesh,
        scratch_types=[pltpu.VMEM_SHARED((ACC_ROWS, CHUNK), jnp.float32),
                       pltpu.VMEM((TB, CHUNK), jnp.float32),
                       pltpu.VMEM((TB,), jnp.int32),
                       pltpu.VMEM((TB,), jnp.int32),
                       pltpu.SemaphoreType.DMA],
    )
    return k(table, srck, dst_pad, zerosC)


def _dinv_body(a_ref, b_ref, o_ref):
    d = a_ref[...] + b_ref[...]
    o_ref[...] = lax.rsqrt(jnp.maximum(d, 1.0))


def _dinv(deg_a, deg_b):
    return pl.pallas_call(
        _dinv_body,
        out_shape=jax.ShapeDtypeStruct((NP, CHUNK), jnp.float32),
    )(deg_a, deg_b)


def _scale_body(x_ref, d_ref, o_ref):
    o_ref[...] = x_ref[...] * d_ref[...]


def _scale(x, d):
    m, k = x.shape
    bm = 640
    return pl.pallas_call(
        _scale_body,
        grid=(m // bm,),
        in_specs=[pl.BlockSpec((bm, k), lambda i: (i, 0)),
                  pl.BlockSpec((bm, 1), lambda i: (i, 0))],
        out_specs=pl.BlockSpec((bm, k), lambda i: (i, 0)),
        out_shape=jax.ShapeDtypeStruct((m, k), jnp.float32),
    )(x, d)


def _mm_body(x_ref, x2_ref, w_ref, b_ref, pre_ref, post_ref, o_ref, *, leaky,
             use_pre, use_post, use_x2):
    x = x_ref[...]
    if use_x2:
        x = x + x2_ref[...]
    if use_pre:
        x = x * pre_ref[...]
    z = jnp.dot(x, w_ref[...], preferred_element_type=jnp.float32)
    z = z + b_ref[...]
    if leaky:
        z = jnp.where(z > 0, z, 0.01 * z)
    if use_post:
        z = z * post_ref[...]
    o_ref[...] = z


def _mm(x, w, b, pre=None, post=None, leaky=True, x2=None):
    m, k = x.shape
    _, n = w.shape
    use_pre = pre is not None
    use_post = post is not None
    use_x2 = x2 is not None
    if pre is None:
        pre = jnp.zeros((m, 1), jnp.float32)
    if post is None:
        post = jnp.zeros((m, 1), jnp.float32)
    if x2 is None:
        x2 = jnp.zeros((m, 1), jnp.float32)
        x2_spec = pl.BlockSpec((_BM, 1), lambda i: (i, 0))
    else:
        x2_spec = pl.BlockSpec((_BM, k), lambda i: (i, 0))
    grid = (m // _BM,)
    return pl.pallas_call(
        functools.partial(_mm_body, leaky=leaky, use_pre=use_pre,
                          use_post=use_post, use_x2=use_x2),
        grid=grid,
        in_specs=[
            pl.BlockSpec((_BM, k), lambda i: (i, 0)),
            x2_spec,
            pl.BlockSpec((k, n), lambda i: (0, 0)),
            pl.BlockSpec((1, n), lambda i: (0, 0)),
            pl.BlockSpec((_BM, 1), lambda i: (i, 0)),
            pl.BlockSpec((_BM, 1), lambda i: (i, 0)),
        ],
        out_specs=pl.BlockSpec((_BM, n), lambda i: (i, 0)),
        out_shape=jax.ShapeDtypeStruct((m, n), jnp.float32),
    )(x, x2, w, b.reshape(1, n), pre, post)


def _pool_body(b_ref, h_ref, o_ref):
    i = pl.program_id(0)

    @pl.when(i == 0)
    def _():
        o_ref[...] = jnp.full((B, H), -jnp.inf, jnp.float32)

    bid = b_ref[...]
    h = h_ref[...]
    lo = bid[0, 0]
    hi = bid[_BM - 1, 0]

    def body(s, c):
        m = bid == s
        v = jnp.max(jnp.where(m, h, -jnp.inf), axis=0, keepdims=True)
        cur = o_ref[pl.ds(s, 1), :]
        o_ref[pl.ds(s, 1), :] = jnp.maximum(cur, v)
        return c

    lax.fori_loop(lo, hi + 1, body, 0)


def _pool(batch2d, h):
    return pl.pallas_call(
        _pool_body,
        grid=(N // _BM,),
        in_specs=[pl.BlockSpec((_BM, 1), lambda i: (i, 0)),
                  pl.BlockSpec((_BM, H), lambda i: (i, 0))],
        out_specs=pl.BlockSpec((B, H), lambda i: (0, 0)),
        out_shape=jax.ShapeDtypeStruct((B, H), jnp.float32),
    )(batch2d, h)


def _head_body(g_ref, w1_ref, b1_ref, w2_ref, b2_ref, o_ref):
    g = g_ref[...]
    g = jnp.where(jnp.isfinite(g), g, 0.0)
    z = jnp.dot(g, w1_ref[...], preferred_element_type=jnp.float32) + b1_ref[...]
    z = jnp.where(z > 0, z, 0.01 * z)
    z = jnp.dot(z, w2_ref[...], preferred_element_type=jnp.float32) + b2_ref[...]
    o_ref[...] = jnp.where(z > 0, z, 0.01 * z)


def _head(g, lin1_W, lin1_b, lin2_W, lin2_b):
    return pl.pallas_call(
        _head_body,
        out_shape=jax.ShapeDtypeStruct((B, 4), jnp.float32),
    )(g, lin1_W, lin1_b.reshape(1, -1), lin2_W, lin2_b.reshape(1, -1))


def kernel(x, edge_index, batch, embed, W1, b1, W2, b2, lin1_W, lin1_b,
           lin2_W, lin2_b):
    ei = edge_index.astype(jnp.int32)
    loop = jnp.arange(N, dtype=jnp.int32)
    src = jnp.concatenate([ei[0], loop])
    dst = jnp.concatenate([ei[1], loop])
    pad_e = EP - (E + N)
    src_pad = jnp.concatenate([src, jnp.zeros((pad_e,), jnp.int32)])
    dst_pad = jnp.concatenate([dst, jnp.full((pad_e,), NP, jnp.int32)])
    x_pad = jnp.concatenate(
        [x.astype(jnp.int32), jnp.zeros((NP - N,), jnp.int32)])

    ones128 = jnp.ones((TB, CHUNK), jnp.float32)
    zerosC = jnp.zeros((ACC_ROWS, CHUNK), jnp.float32)

    deg_a, deg_b = _sc_degree(dst_pad, ones128, zerosC)
    dinvC = _dinv(deg_a, deg_b)
    dinv = dinvC[:, :1]
    dinv_n = dinv[:N]

    embed_pad = jnp.pad(embed, ((0, 0), (0, D_PAD - D_EMB)))
    h0 = _sc_gather(embed_pad, x_pad)
    s0 = _scale(h0, dinv)
    k1 = D_PAD // CHUNK
    srck1 = (src_pad[None, None, :] * k1
             + jnp.arange(k1, dtype=jnp.int32)[:, None, None])
    a1a, a1b = _sc_segsum(s0.reshape(NP * k1, CHUNK), srck1, dst_pad, zerosC,
                          nchunk=k1)
    a1a = jnp.transpose(a1a, (1, 0, 2)).reshape(NP, D_PAD)
    a1b = jnp.transpose(a1b, (1, 0, 2)).reshape(NP, D_PAD)
    W1p = jnp.pad(W1, ((0, D_PAD - D_EMB), (0, H_PAD - H)))
    b1p = jnp.pad(b1, (0, H_PAD - H))
    s1 = _mm(a1a[:N], W1p, b1p, pre=dinv_n, post=dinv_n, leaky=True,
             x2=a1b[:N])

    k2 = H_PAD // CHUNK
    srck2 = (src_pad[None, None, :] * k2
             + jnp.arange(k2, dtype=jnp.int32)[:, None, None])
    a2a, a2b = _sc_segsum(s1.reshape(N * k2, CHUNK), srck2, dst_pad, zerosC,
                          nchunk=k2)
    a2a = jnp.transpose(a2a, (1, 0, 2)).reshape(NP, H_PAD)
    a2b = jnp.transpose(a2b, (1, 0, 2)).reshape(NP, H_PAD)
    W2p = jnp.pad(W2, ((0, H_PAD - H), (0, 0)))
    h2 = _mm(a2a[:N], W2p, b2, pre=dinv_n, leaky=True, x2=a2b[:N])

    g = _pool(batch.astype(jnp.int32).reshape(N, 1), h2)
    return _head(g, lin1_W, lin1_b, lin2_W, lin2_b)
